# Initial kernel scaffold; baseline (speedup 1.0000x reference)
import jax
import jax.numpy as jnp
from jax.experimental import pallas as pl


def kernel(xi, xv, tables, W1, b1, W2, b2, W3, b3):
    raise NotImplementedError("write your pallas kernel here")


# speedup vs baseline: 1.9903x; 1.9903x over previous
import functools

import jax
import jax.numpy as jnp
from jax import lax
from jax.experimental import pallas as pl
from jax.experimental.pallas import tpu as pltpu
from jax.experimental.pallas import tpu_sc as plsc

_NC = 2
_NS = 16
_NW = _NC * _NS

_ROWS_PER_STREAM = 128
_STREAMS_PER_SUPER = 8


def _sc_gather(table_flat, idx):
    n_rows, d = table_flat.shape
    nw, groups, rps = idx.shape
    assert nw == _NW and rps == _ROWS_PER_STREAM
    assert groups % _STREAMS_PER_SUPER == 0
    supers = groups // _STREAMS_PER_SUPER
    rows_per_super = _STREAMS_PER_SUPER * rps
    ipw = groups * rps
    total = nw * ipw

    mesh = plsc.VectorSubcoreMesh(
        core_axis_name="c", subcore_axis_name="s",
        num_cores=_NC, num_subcores=_NS)

    @functools.partial(
        pl.kernel,
        mesh=mesh,
        compiler_params=pltpu.CompilerParams(use_tc_tiling_on_sc=False),
        out_type=jax.ShapeDtypeStruct((total, d), jnp.float32),
        scratch_types=[
            pltpu.VMEM((groups, rps), jnp.int32),
            pltpu.VMEM((rows_per_super, d), jnp.float32),
            pltpu.SemaphoreType.DMA,
        ],
    )
    def gather_kernel(tbl_hbm, idx_hbm, out_hbm, idx_v, rows_v, sem):
        wid = lax.axis_index("s") * _NC + lax.axis_index("c")
        base = wid * ipw
        pltpu.sync_copy(idx_hbm.at[wid], idx_v)

        @pl.loop(0, supers)
        def _super(sp):
            cps = []
            for j in range(_STREAMS_PER_SUPER):
                g = sp * _STREAMS_PER_SUPER + j
                cps.append(pltpu.async_copy(
                    tbl_hbm.at[idx_v.at[g]],
                    rows_v.at[pl.ds(j * rps, rps)],
                    sem))
            for cp in cps:
                cp.wait()
            pltpu.sync_copy(
                rows_v, out_hbm.at[pl.ds(base + sp * rows_per_super, rows_per_super)])

    return gather_kernel(table_flat, idx)


def _mlp_body(xi_ref, xe_ref, w1a_ref, w1b_ref, w2_ref, w3_ref,
              b1_ref, b2_ref, b3_ref, o_ref):
    x1 = jnp.dot(xe_ref[...], w1b_ref[...], preferred_element_type=jnp.float32)
    x1 = x1 + jnp.dot(xi_ref[...], w1a_ref[...], preferred_element_type=jnp.float32)
    h1 = jnp.maximum(x1 + b1_ref[...], 0.0)
    h2 = jnp.maximum(
        jnp.dot(h1, w2_ref[...], preferred_element_type=jnp.float32) + b2_ref[...], 0.0)
    o = jnp.dot(h2, w3_ref[...], preferred_element_type=jnp.float32) + b3_ref[...]
    o_ref[...] = jax.nn.sigmoid(o)


def _mlp(xi, xe, w1a, w1b, w2, w3, b1, b2, b3, tile_b=1024):
    b, f_cont = xi.shape
    _, e_dim = xe.shape
    h1 = w2.shape[0]
    grid = (b // tile_b,)
    full = lambda shape: pl.BlockSpec(shape, lambda i: (0, 0))
    return pl.pallas_call(
        _mlp_body,
        grid=grid,
        in_specs=[
            pl.BlockSpec((tile_b, f_cont), lambda i: (i, 0)),
            pl.BlockSpec((tile_b, e_dim), lambda i: (i, 0)),
            full(w1a.shape),
            full(w1b.shape),
            full(w2.shape),
            full(w3.shape),
            full(b1.shape),
            full(b2.shape),
            full(b3.shape),
        ],
        out_specs=pl.BlockSpec((tile_b, 1), lambda i: (i, 0)),
        out_shape=jax.ShapeDtypeStruct((b, 1), jnp.float32),
    )(xi, xe, w1a, w1b, w2, w3, b1, b2, b3)


def kernel(xi, xv, tables, W1, b1, W2, b2, W3, b3):
    b, f_cat = xv.shape
    f, v, d = tables.shape
    f_cont = xi.shape[1]
    idx = xv.astype(jnp.int32) + (jnp.arange(f, dtype=jnp.int32) * v)[None, :]
    ipw = (b * f_cat) // _NW
    idx = idx.reshape(_NW, ipw // _ROWS_PER_STREAM, _ROWS_PER_STREAM)
    xe = _sc_gather(tables.reshape(f * v, d), idx)
    xe = xe.reshape(b, f_cat * d)
    return _mlp(
        xi, xe,
        W1[:f_cont], W1[f_cont:], W2, W3,
        b1.reshape(1, -1), b2.reshape(1, -1), b3.reshape(1, 1))
